# Initial kernel scaffold; baseline (speedup 1.0000x reference)
import jax
import jax.numpy as jnp
from jax.experimental import pallas as pl


def kernel(embs_local_global, cu_seqlens, Wq, Wk, Wv, bq, bk, bv):
    raise NotImplementedError("write your pallas kernel here")


# speedup vs baseline: 17.0417x; 17.0417x over previous
import jax
import jax.numpy as jnp
from jax.experimental import pallas as pl
from jax.experimental.pallas import tpu as pltpu

_L = 512


def _attn_kernel(cu_ref, embs_ref, wq_ref, wk_ref, wv_ref, bq_ref, bk_ref,
                 bv_ref, out_ref):
    b = pl.program_id(0)
    t = embs_ref.shape[0]
    start = cu_ref[b]
    end = cu_ref[b + 1]
    sc = jnp.minimum(start, t - _L)

    x = embs_ref[pl.ds(sc, _L), :]
    q = jnp.dot(x, wq_ref[...], preferred_element_type=jnp.float32) + bq_ref[...]
    k = jnp.dot(x, wk_ref[...], preferred_element_type=jnp.float32) + bk_ref[...]
    v = jnp.dot(x, wv_ref[...], preferred_element_type=jnp.float32) + bv_ref[...]

    row_g = sc + jax.lax.broadcasted_iota(jnp.int32, (_L, _L), 0)
    col_g = sc + jax.lax.broadcasted_iota(jnp.int32, (_L, _L), 1)

    s = jax.lax.dot_general(q, k, (((1,), (1,)), ((), ())),
                            preferred_element_type=jnp.float32)
    mask = (col_g >= start) & (col_g < end) & (col_g != row_g)
    s = jnp.where(mask, s, -1e30)
    m = jnp.max(s, axis=1, keepdims=True)
    p = jnp.exp(s - m)
    attn = p / jnp.sum(p, axis=1, keepdims=True)
    o = jnp.dot(attn, v, preferred_element_type=jnp.float32)

    row1 = sc + jax.lax.broadcasted_iota(jnp.int32, (_L, 1), 0)
    row_valid = (row1 >= start) & (row1 < end)
    cur = out_ref[pl.ds(sc, _L), :]
    out_ref[pl.ds(sc, _L), :] = jnp.where(row_valid, o, cur)


def kernel(embs_local_global, cu_seqlens, Wq, Wk, Wv, bq, bk, bv):
    t, d = embs_local_global.shape
    nseg = cu_seqlens.shape[0] - 1
    bq2 = bq.reshape(1, d)
    bk2 = bk.reshape(1, d)
    bv2 = bv.reshape(1, d)
    full = lambda shape: pl.BlockSpec(shape, lambda b: (0,) * len(shape))
    return pl.pallas_call(
        _attn_kernel,
        grid=(nseg,),
        in_specs=[
            pl.BlockSpec(memory_space=pltpu.SMEM),
            full((t, d)),
            full((d, d)),
            full((d, d)),
            full((d, d)),
            full((1, d)),
            full((1, d)),
            full((1, d)),
        ],
        out_specs=full((t, d)),
        out_shape=jax.ShapeDtypeStruct((t, d), jnp.float32),
        compiler_params=pltpu.CompilerParams(
            dimension_semantics=("arbitrary",)),
    )(cu_seqlens, embs_local_global, Wq, Wk, Wv, bq2, bk2, bv2)
